# baseline (speedup 1.0000x reference)
import functools

import jax
import jax.numpy as jnp
from jax import lax
from jax.experimental import pallas as pl
from jax.experimental.pallas import tpu as pltpu
from jax.experimental.pallas import tpu_sc as plsc

NUM_FIELDS = 26
VOCAB = 100000
EMBED = 32
BATCH = 4096
OUT_D = NUM_FIELDS * EMBED

_NC = 2
_NS = 16
_NW = _NC * _NS
_BPW = BATCH // _NW
_NH = 2
_RH = _BPW // _NH
_L = 16
_GS = 4
_NG = (NUM_FIELDS + _GS - 1) // _GS
_LB = 128
_NLB = (VOCAB + _LB - 1) // _LB
_NSLOT = 4


def _tc_pack_group(ws_t):
    n = len(ws_t)

    def body(*refs):
        ins = refs[:n]
        o = refs[n]
        cols = [r[...].T for r in ins]
        if n < _GS:
            cols.append(jnp.zeros((_LB, (_GS - n) * EMBED), jnp.float32))
        o[...] = jnp.concatenate(cols, axis=1)

    return pl.pallas_call(
        body,
        grid=(_NLB,),
        in_specs=[pl.BlockSpec((EMBED, _LB), lambda g: (0, g))
                  for _ in range(n)],
        out_specs=pl.BlockSpec((_LB, _GS * EMBED), lambda g: (g, 0)),
        out_shape=jax.ShapeDtypeStruct((VOCAB, _GS * EMBED), jnp.float32),
    )(*ws_t)


def _sc_embed(feats, packs):
    mesh = plsc.VectorSubcoreMesh(core_axis_name="c", subcore_axis_name="s")

    @functools.partial(
        pl.kernel,
        mesh=mesh,
        out_type=jax.ShapeDtypeStruct((BATCH, OUT_D), jnp.float32),
        scratch_types=[
            pltpu.VMEM((_NH * NUM_FIELDS, _RH), jnp.int32),
            pltpu.VMEM((_NSLOT, _RH, _GS * EMBED), jnp.float32),
            pltpu.VMEM((_RH, OUT_D), jnp.float32),
            pltpu.SemaphoreType.DMA,
        ] + [pltpu.SemaphoreType.DMA for _ in range(_NSLOT)],
        compiler_params=pltpu.CompilerParams(needs_layout_passes=False),
    )
    def k(*refs):
        fs = refs[:NUM_FIELDS]
        ws = refs[NUM_FIELDS:NUM_FIELDS + _NG]
        rest = refs[NUM_FIELDS + _NG:]
        out_hbm, idxs, rows, tile_v, sem_i = rest[:5]
        sem_g = rest[5:5 + _NSLOT]
        wid = lax.axis_index("s") * _NC + lax.axis_index("c")
        base = wid * _BPW

        idx_cps = []
        for h in range(_NH):
            for i in range(NUM_FIELDS):
                idx_cps.append(pltpu.async_copy(
                    fs[i].at[pl.ds(base + h * _RH, _RH)],
                    idxs.at[h * NUM_FIELDS + i], sem_i))
        for c in idx_cps:
            c.wait()

        for h in range(_NH):
            cps = {}

            def fire(i):
                g = i // _GS
                s = i % _NSLOT
                cps[i] = pltpu.async_copy(
                    ws[g].at[idxs.at[h * NUM_FIELDS + i]],
                    rows.at[s], sem_g[s])

            for i in range(_NSLOT):
                fire(i)
            for i in range(NUM_FIELDS):
                cps[i].wait()
                s = i % _NSLOT
                f = i % _GS

                def row_body(r, _):
                    for c in range(0, EMBED, _L):
                        tile_v[r, pl.ds(i * EMBED + c, _L)] = (
                            rows[s, r, pl.ds(f * EMBED + c, _L)])
                    return ()
                jax.lax.fori_loop(0, _RH, row_body, ())
                if i + _NSLOT < NUM_FIELDS:
                    fire(i + _NSLOT)

            pltpu.sync_copy(
                tile_v, out_hbm.at[pl.ds(base + h * _RH, _RH), :])

    return k(*feats, *packs)


def kernel(feat_0, feat_1, feat_2, feat_3, feat_4, feat_5, feat_6, feat_7, feat_8, feat_9, feat_10, feat_11, feat_12, feat_13, feat_14, feat_15, feat_16, feat_17, feat_18, feat_19, feat_20, feat_21, feat_22, feat_23, feat_24, feat_25, W_0, W_1, W_2, W_3, W_4, W_5, W_6, W_7, W_8, W_9, W_10, W_11, W_12, W_13, W_14, W_15, W_16, W_17, W_18, W_19, W_20, W_21, W_22, W_23, W_24, W_25):
    feats = [feat_0, feat_1, feat_2, feat_3, feat_4, feat_5, feat_6, feat_7, feat_8, feat_9, feat_10, feat_11, feat_12, feat_13, feat_14, feat_15, feat_16, feat_17, feat_18, feat_19, feat_20, feat_21, feat_22, feat_23, feat_24, feat_25]
    tables = [W_0, W_1, W_2, W_3, W_4, W_5, W_6, W_7, W_8, W_9, W_10, W_11, W_12, W_13, W_14, W_15, W_16, W_17, W_18, W_19, W_20, W_21, W_22, W_23, W_24, W_25]
    packs = []
    for g in range(_NG):
        grp = tables[g * _GS:(g + 1) * _GS]
        packs.append(_tc_pack_group([w.T for w in grp]))
    return _sc_embed(feats, packs)


# speedup vs baseline: 4.4710x; 4.4710x over previous
import functools

import jax
import jax.numpy as jnp
from jax import lax
from jax.experimental import pallas as pl
from jax.experimental.pallas import tpu as pltpu
from jax.experimental.pallas import tpu_sc as plsc

NUM_FIELDS = 26
VOCAB = 100000
EMBED = 32
BATCH = 4096
OUT_D = NUM_FIELDS * EMBED

_NC = 2
_NS = 16
_NW = _NC * _NS
_BPW = BATCH // _NW
_NH = 2
_RH = _BPW // _NH
_L = 16
_GS = 4
_NG = (NUM_FIELDS + _GS - 1) // _GS
_LB = 512
_NLB = (VOCAB + _LB - 1) // _LB
_NSLOT = 4


def _tc_pack_all(ws_t):
    n = len(ws_t)

    def body(*refs):
        ins = refs[:n]
        outs = refs[n:]
        for g in range(_NG):
            grp = ins[g * _GS:(g + 1) * _GS]
            cols = [r[...].T for r in grp]
            if len(grp) < _GS:
                cols.append(jnp.zeros(
                    (_LB, (_GS - len(grp)) * EMBED), jnp.float32))
            outs[g][...] = jnp.concatenate(cols, axis=1)

    return pl.pallas_call(
        body,
        grid=(_NLB,),
        in_specs=[pl.BlockSpec((EMBED, _LB), lambda g: (0, g))
                  for _ in range(n)],
        out_specs=[pl.BlockSpec((_LB, _GS * EMBED), lambda g: (g, 0))
                   for _ in range(_NG)],
        out_shape=[jax.ShapeDtypeStruct((VOCAB, _GS * EMBED), jnp.float32)
                   for _ in range(_NG)],
    )(*ws_t)


def _sc_embed(feats, packs):
    mesh = plsc.VectorSubcoreMesh(core_axis_name="c", subcore_axis_name="s")

    @functools.partial(
        pl.kernel,
        mesh=mesh,
        out_type=jax.ShapeDtypeStruct((BATCH, OUT_D), jnp.float32),
        scratch_types=[
            pltpu.VMEM((_NH * NUM_FIELDS, _RH), jnp.int32),
            pltpu.VMEM((_NSLOT, _RH, _GS * EMBED), jnp.float32),
            pltpu.VMEM((_RH, OUT_D), jnp.float32),
            pltpu.SemaphoreType.DMA,
        ] + [pltpu.SemaphoreType.DMA for _ in range(_NSLOT)],
        compiler_params=pltpu.CompilerParams(needs_layout_passes=False),
    )
    def k(*refs):
        fs = refs[:NUM_FIELDS]
        ws = refs[NUM_FIELDS:NUM_FIELDS + _NG]
        rest = refs[NUM_FIELDS + _NG:]
        out_hbm, idxs, rows, tile_v, sem_i = rest[:5]
        sem_g = rest[5:5 + _NSLOT]
        wid = lax.axis_index("s") * _NC + lax.axis_index("c")
        base = wid * _BPW

        idx_cps = []
        for h in range(_NH):
            for i in range(NUM_FIELDS):
                idx_cps.append(pltpu.async_copy(
                    fs[i].at[pl.ds(base + h * _RH, _RH)],
                    idxs.at[h * NUM_FIELDS + i], sem_i))
        for c in idx_cps:
            c.wait()

        for h in range(_NH):
            cps = {}

            def fire(i):
                g = i // _GS
                s = i % _NSLOT
                cps[i] = pltpu.async_copy(
                    ws[g].at[idxs.at[h * NUM_FIELDS + i]],
                    rows.at[s], sem_g[s])

            for i in range(_NSLOT):
                fire(i)
            for i in range(NUM_FIELDS):
                cps[i].wait()
                s = i % _NSLOT
                f = i % _GS

                def row_body(r, _):
                    for c in range(0, EMBED, _L):
                        tile_v[r, pl.ds(i * EMBED + c, _L)] = (
                            rows[s, r, pl.ds(f * EMBED + c, _L)])
                    return ()
                jax.lax.fori_loop(0, _RH, row_body, ())
                if i + _NSLOT < NUM_FIELDS:
                    fire(i + _NSLOT)

            pltpu.sync_copy(
                tile_v, out_hbm.at[pl.ds(base + h * _RH, _RH), :])

    return k(*feats, *packs)


def kernel(feat_0, feat_1, feat_2, feat_3, feat_4, feat_5, feat_6, feat_7, feat_8, feat_9, feat_10, feat_11, feat_12, feat_13, feat_14, feat_15, feat_16, feat_17, feat_18, feat_19, feat_20, feat_21, feat_22, feat_23, feat_24, feat_25, W_0, W_1, W_2, W_3, W_4, W_5, W_6, W_7, W_8, W_9, W_10, W_11, W_12, W_13, W_14, W_15, W_16, W_17, W_18, W_19, W_20, W_21, W_22, W_23, W_24, W_25):
    feats = [feat_0, feat_1, feat_2, feat_3, feat_4, feat_5, feat_6, feat_7, feat_8, feat_9, feat_10, feat_11, feat_12, feat_13, feat_14, feat_15, feat_16, feat_17, feat_18, feat_19, feat_20, feat_21, feat_22, feat_23, feat_24, feat_25]
    tables = [W_0, W_1, W_2, W_3, W_4, W_5, W_6, W_7, W_8, W_9, W_10, W_11, W_12, W_13, W_14, W_15, W_16, W_17, W_18, W_19, W_20, W_21, W_22, W_23, W_24, W_25]
    packs = _tc_pack_all([w.T for w in tables])
    return _sc_embed(feats, packs)


# speedup vs baseline: 4.6491x; 1.0398x over previous
import functools

import jax
import jax.numpy as jnp
from jax import lax
from jax.experimental import pallas as pl
from jax.experimental.pallas import tpu as pltpu
from jax.experimental.pallas import tpu_sc as plsc

NUM_FIELDS = 26
VOCAB = 100000
EMBED = 32
BATCH = 4096
OUT_D = NUM_FIELDS * EMBED

_NC = 2
_NS = 16
_NW = _NC * _NS
_BPW = BATCH // _NW
_NH = 2
_RH = _BPW // _NH
_L = 16
_GS = 4
_NG = (NUM_FIELDS + _GS - 1) // _GS
_LB = 1024
_NLB = (VOCAB + _LB - 1) // _LB
_NSLOT = 4


def _tc_pack_all(ws_t):
    n = len(ws_t)

    def body(*refs):
        ins = refs[:n]
        outs = refs[n:]
        for g in range(_NG):
            grp = ins[g * _GS:(g + 1) * _GS]
            cols = [r[...].T for r in grp]
            if len(grp) < _GS:
                cols.append(jnp.zeros(
                    (_LB, (_GS - len(grp)) * EMBED), jnp.float32))
            outs[g][...] = jnp.concatenate(cols, axis=1)

    return pl.pallas_call(
        body,
        grid=(_NLB,),
        in_specs=[pl.BlockSpec((EMBED, _LB), lambda g: (0, g))
                  for _ in range(n)],
        out_specs=[pl.BlockSpec((_LB, _GS * EMBED), lambda g: (g, 0))
                   for _ in range(_NG)],
        out_shape=[jax.ShapeDtypeStruct((VOCAB, _GS * EMBED), jnp.float32)
                   for _ in range(_NG)],
    )(*ws_t)


def _sc_embed(feats, packs):
    mesh = plsc.VectorSubcoreMesh(core_axis_name="c", subcore_axis_name="s")

    @functools.partial(
        pl.kernel,
        mesh=mesh,
        out_type=jax.ShapeDtypeStruct((BATCH, OUT_D), jnp.float32),
        scratch_types=[
            pltpu.VMEM((_NH * NUM_FIELDS, _RH), jnp.int32),
            pltpu.VMEM((_NSLOT, _RH, _GS * EMBED), jnp.float32),
            pltpu.VMEM((_RH, OUT_D), jnp.float32),
            pltpu.SemaphoreType.DMA,
        ] + [pltpu.SemaphoreType.DMA for _ in range(_NSLOT)],
        compiler_params=pltpu.CompilerParams(needs_layout_passes=False),
    )
    def k(*refs):
        fs = refs[:NUM_FIELDS]
        ws = refs[NUM_FIELDS:NUM_FIELDS + _NG]
        rest = refs[NUM_FIELDS + _NG:]
        out_hbm, idxs, rows, tile_v, sem_i = rest[:5]
        sem_g = rest[5:5 + _NSLOT]
        wid = lax.axis_index("s") * _NC + lax.axis_index("c")
        base = wid * _BPW

        idx_cps = []
        for h in range(_NH):
            for i in range(NUM_FIELDS):
                idx_cps.append(pltpu.async_copy(
                    fs[i].at[pl.ds(base + h * _RH, _RH)],
                    idxs.at[h * NUM_FIELDS + i], sem_i))
        for c in idx_cps:
            c.wait()

        for h in range(_NH):
            cps = {}

            def fire(i):
                g = i // _GS
                s = i % _NSLOT
                cps[i] = pltpu.async_copy(
                    ws[g].at[idxs.at[h * NUM_FIELDS + i]],
                    rows.at[s], sem_g[s])

            for i in range(_NSLOT):
                fire(i)
            for i in range(NUM_FIELDS):
                cps[i].wait()
                s = i % _NSLOT
                f = i % _GS

                def row_body(r, _):
                    for c in range(0, EMBED, _L):
                        tile_v[r, pl.ds(i * EMBED + c, _L)] = (
                            rows[s, r, pl.ds(f * EMBED + c, _L)])
                    return ()
                jax.lax.fori_loop(0, _RH, row_body, ())
                if i + _NSLOT < NUM_FIELDS:
                    fire(i + _NSLOT)

            pltpu.sync_copy(
                tile_v, out_hbm.at[pl.ds(base + h * _RH, _RH), :])

    return k(*feats, *packs)


def kernel(feat_0, feat_1, feat_2, feat_3, feat_4, feat_5, feat_6, feat_7, feat_8, feat_9, feat_10, feat_11, feat_12, feat_13, feat_14, feat_15, feat_16, feat_17, feat_18, feat_19, feat_20, feat_21, feat_22, feat_23, feat_24, feat_25, W_0, W_1, W_2, W_3, W_4, W_5, W_6, W_7, W_8, W_9, W_10, W_11, W_12, W_13, W_14, W_15, W_16, W_17, W_18, W_19, W_20, W_21, W_22, W_23, W_24, W_25):
    feats = [feat_0, feat_1, feat_2, feat_3, feat_4, feat_5, feat_6, feat_7, feat_8, feat_9, feat_10, feat_11, feat_12, feat_13, feat_14, feat_15, feat_16, feat_17, feat_18, feat_19, feat_20, feat_21, feat_22, feat_23, feat_24, feat_25]
    tables = [W_0, W_1, W_2, W_3, W_4, W_5, W_6, W_7, W_8, W_9, W_10, W_11, W_12, W_13, W_14, W_15, W_16, W_17, W_18, W_19, W_20, W_21, W_22, W_23, W_24, W_25]
    packs = _tc_pack_all([w.T for w in tables])
    return _sc_embed(feats, packs)


# speedup vs baseline: 4.7707x; 1.0261x over previous
import functools

import jax
import jax.numpy as jnp
from jax import lax
from jax.experimental import pallas as pl
from jax.experimental.pallas import tpu as pltpu
from jax.experimental.pallas import tpu_sc as plsc

NUM_FIELDS = 26
VOCAB = 100000
EMBED = 32
BATCH = 4096
OUT_D = NUM_FIELDS * EMBED

_NC = 2
_NS = 16
_NW = _NC * _NS
_BPW = BATCH // _NW
_NH = 2
_RH = _BPW // _NH
_L = 16
_GS = 4
_NG = (NUM_FIELDS + _GS - 1) // _GS
_LB = 2048
_NLB = (VOCAB + _LB - 1) // _LB
_NSLOT = 4


def _tc_pack_all(ws_t):
    n = len(ws_t)

    def body(*refs):
        ins = refs[:n]
        outs = refs[n:]
        for g in range(_NG):
            grp = ins[g * _GS:(g + 1) * _GS]
            cols = [r[...].T for r in grp]
            if len(grp) < _GS:
                cols.append(jnp.zeros(
                    (_LB, (_GS - len(grp)) * EMBED), jnp.float32))
            outs[g][...] = jnp.concatenate(cols, axis=1)

    return pl.pallas_call(
        body,
        grid=(_NLB,),
        in_specs=[pl.BlockSpec((EMBED, _LB), lambda g: (0, g))
                  for _ in range(n)],
        out_specs=[pl.BlockSpec((_LB, _GS * EMBED), lambda g: (g, 0))
                   for _ in range(_NG)],
        out_shape=[jax.ShapeDtypeStruct((VOCAB, _GS * EMBED), jnp.float32)
                   for _ in range(_NG)],
    )(*ws_t)


def _sc_embed(feats, packs):
    mesh = plsc.VectorSubcoreMesh(core_axis_name="c", subcore_axis_name="s")

    @functools.partial(
        pl.kernel,
        mesh=mesh,
        out_type=jax.ShapeDtypeStruct((BATCH, OUT_D), jnp.float32),
        scratch_types=[
            pltpu.VMEM((_NH * NUM_FIELDS, _RH), jnp.int32),
            pltpu.VMEM((_NSLOT, _RH, _GS * EMBED), jnp.float32),
            pltpu.VMEM((_RH, OUT_D), jnp.float32),
            pltpu.SemaphoreType.DMA,
        ] + [pltpu.SemaphoreType.DMA for _ in range(_NSLOT)],
        compiler_params=pltpu.CompilerParams(needs_layout_passes=False),
    )
    def k(*refs):
        fs = refs[:NUM_FIELDS]
        ws = refs[NUM_FIELDS:NUM_FIELDS + _NG]
        rest = refs[NUM_FIELDS + _NG:]
        out_hbm, idxs, rows, tile_v, sem_i = rest[:5]
        sem_g = rest[5:5 + _NSLOT]
        wid = lax.axis_index("s") * _NC + lax.axis_index("c")
        base = wid * _BPW

        idx_cps = []
        for h in range(_NH):
            for i in range(NUM_FIELDS):
                idx_cps.append(pltpu.async_copy(
                    fs[i].at[pl.ds(base + h * _RH, _RH)],
                    idxs.at[h * NUM_FIELDS + i], sem_i))
        for c in idx_cps:
            c.wait()

        for h in range(_NH):
            cps = {}

            def fire(i):
                g = i // _GS
                s = i % _NSLOT
                cps[i] = pltpu.async_copy(
                    ws[g].at[idxs.at[h * NUM_FIELDS + i]],
                    rows.at[s], sem_g[s])

            for i in range(_NSLOT):
                fire(i)
            for i in range(NUM_FIELDS):
                cps[i].wait()
                s = i % _NSLOT
                f = i % _GS

                def row_body(r, _):
                    for c in range(0, EMBED, _L):
                        tile_v[r, pl.ds(i * EMBED + c, _L)] = (
                            rows[s, r, pl.ds(f * EMBED + c, _L)])
                    return ()
                jax.lax.fori_loop(0, _RH, row_body, ())
                if i + _NSLOT < NUM_FIELDS:
                    fire(i + _NSLOT)

            pltpu.sync_copy(
                tile_v, out_hbm.at[pl.ds(base + h * _RH, _RH), :])

    return k(*feats, *packs)


def kernel(feat_0, feat_1, feat_2, feat_3, feat_4, feat_5, feat_6, feat_7, feat_8, feat_9, feat_10, feat_11, feat_12, feat_13, feat_14, feat_15, feat_16, feat_17, feat_18, feat_19, feat_20, feat_21, feat_22, feat_23, feat_24, feat_25, W_0, W_1, W_2, W_3, W_4, W_5, W_6, W_7, W_8, W_9, W_10, W_11, W_12, W_13, W_14, W_15, W_16, W_17, W_18, W_19, W_20, W_21, W_22, W_23, W_24, W_25):
    feats = [feat_0, feat_1, feat_2, feat_3, feat_4, feat_5, feat_6, feat_7, feat_8, feat_9, feat_10, feat_11, feat_12, feat_13, feat_14, feat_15, feat_16, feat_17, feat_18, feat_19, feat_20, feat_21, feat_22, feat_23, feat_24, feat_25]
    tables = [W_0, W_1, W_2, W_3, W_4, W_5, W_6, W_7, W_8, W_9, W_10, W_11, W_12, W_13, W_14, W_15, W_16, W_17, W_18, W_19, W_20, W_21, W_22, W_23, W_24, W_25]
    packs = _tc_pack_all([w.T for w in tables])
    return _sc_embed(feats, packs)
